# Initial kernel scaffold; baseline (speedup 1.0000x reference)
import jax
import jax.numpy as jnp
from jax.experimental import pallas as pl


def kernel(lr_batch, gc0_w, gsr_w, gc1_w, gc2_w):
    raise NotImplementedError("write your pallas kernel here")


# speedup vs baseline: 1.0033x; 1.0033x over previous
import jax
import jax.numpy as jnp
from jax import lax
from jax.experimental import pallas as pl
from jax.experimental.pallas import tpu as pltpu

_G = 8


def _diag(n):
    row = lax.broadcasted_iota(jnp.int32, (n, n), 0)
    col = lax.broadcasted_iota(jnp.int32, (n, n), 1)
    return row == col


def _encode_kernel(lr_ref, w0_ref, a_ref, gc_ref):
    f32 = jnp.float32
    bf16 = jnp.bfloat16
    lr = lr_ref[...]
    g, n, _ = lr.shape
    h = w0_ref.shape[1]
    r_col = lax.rsqrt(jnp.sum(lr, axis=2, keepdims=True))
    r_col = jnp.where(jnp.isinf(r_col), 0.0, r_col)
    r_row = lax.rsqrt(jnp.sum(lr, axis=1, keepdims=True))
    r_row = jnp.where(jnp.isinf(r_row), 0.0, r_row)
    a = (r_col * lr) * r_row
    a_ref[...] = a
    a_stack = a.astype(bf16).reshape(g * n, n)
    gc = jnp.dot(a_stack, w0_ref[...].astype(bf16),
                 preferred_element_type=f32)
    gc_ref[...] = jnp.maximum(gc, 0.0).astype(bf16).reshape(g, n, h)


def _decoder_kernel(aw_ref, u_ref, x_ref, w1_ref, w2_ref, z_ref):
    f32 = jnp.float32
    bf16 = jnp.bfloat16
    aw = aw_ref[...].astype(bf16)
    w1 = w1_ref[...].astype(bf16)
    w2 = w2_ref[...].astype(bf16)
    g, n, _ = u_ref.shape
    h = aw.shape[0]
    diag = _diag(h)
    u_stack = u_ref[...].astype(bf16).reshape(g * n, n)
    dn_bt = (((1,), (1,)), ((), ()))
    b_all = lax.dot_general(aw, u_stack, dn_bt,
                            preferred_element_type=f32)
    for i in range(g):
        b = b_all[:, i * n:(i + 1) * n].astype(bf16)
        x = x_ref[i]
        f_d = jnp.abs(jnp.dot(b, x, preferred_element_type=f32))
        adj = jnp.where(diag, 1.0, f_d).astype(bf16)
        xo = lax.dot_general(adj, adj, dn_bt, preferred_element_type=f32)
        z0 = jnp.abs(jnp.where(diag, 1.0, xo)).astype(bf16)
        h1 = jnp.dot(z0, w1, preferred_element_type=f32).astype(bf16)
        h2 = jnp.dot(adj, h1, preferred_element_type=f32)
        h2 = jnp.maximum(h2, 0.0).astype(bf16)
        o1 = jnp.dot(h2, w2, preferred_element_type=f32).astype(bf16)
        o2 = jnp.tanh(jnp.dot(adj, o1, preferred_element_type=f32))
        out = (o2 + o2.T) * 0.5
        z_ref[i] = jnp.where(diag, 0.0, out)


def kernel(lr_batch, gc0_w, gsr_w, gc1_w, gc2_w):
    f32 = jnp.float32
    lr_batch = lr_batch.astype(f32)
    batch, n, _ = lr_batch.shape
    h = gc0_w.shape[1]
    hid = gc1_w.shape[1]
    g = _G if batch % _G == 0 else 1
    steps = batch // g
    par = pltpu.CompilerParams(dimension_semantics=("parallel",))

    a_mat, gc = pl.pallas_call(
        _encode_kernel,
        grid=(steps,),
        out_shape=(jax.ShapeDtypeStruct((batch, n, n), f32),
                   jax.ShapeDtypeStruct((batch, n, h), jnp.bfloat16)),
        in_specs=[pl.BlockSpec((g, n, n), lambda s: (s, 0, 0)),
                  pl.BlockSpec((n, h), lambda s: (0, 0))],
        out_specs=(pl.BlockSpec((g, n, n), lambda s: (s, 0, 0)),
                   pl.BlockSpec((g, n, h), lambda s: (s, 0, 0))),
        compiler_params=par,
    )(lr_batch, gc0_w)

    a_w = gsr_w[:, :n] + gsr_w[:, n:]
    _, u_mat = jnp.linalg.eigh(a_mat, UPLO="U")
    u_mat = u_mat.astype(f32)

    z = pl.pallas_call(
        _decoder_kernel,
        grid=(steps,),
        out_shape=jax.ShapeDtypeStruct((batch, h, h), f32),
        in_specs=[pl.BlockSpec((h, n), lambda s: (0, 0)),
                  pl.BlockSpec((g, n, n), lambda s: (s, 0, 0)),
                  pl.BlockSpec((g, n, h), lambda s: (s, 0, 0)),
                  pl.BlockSpec((h, hid), lambda s: (0, 0)),
                  pl.BlockSpec((hid, h), lambda s: (0, 0))],
        out_specs=pl.BlockSpec((g, h, h), lambda s: (s, 0, 0)),
        compiler_params=par,
    )(a_w, u_mat, gc, gc1_w, gc2_w)
    return z
